# baseline (speedup 1.0000x reference)
import functools

import jax
import jax.numpy as jnp
from jax import lax
from jax.experimental import pallas as pl
from jax.experimental.pallas import tpu as pltpu
from jax.experimental.pallas import tpu_sc as plsc

N = 10000
NPAD = 10112
E = 160000
EPAD = 163840
NTILES = 32
NSUB = 16
NCORE = 2
BATCHES = EPAD // (NTILES * 128)
BE = 128
RPT = NPAD // NSUB
RB = 8
RBS = NPAD // RB

_SC_MESH = plsc.VectorSubcoreMesh(core_axis_name="c", subcore_axis_name="s")


def _srcm_body(src_ref, dst_ref, o_ref):
    s = src_ref[...]
    d = dst_ref[...]
    o_ref[...] = jnp.where(s == d, jnp.int32(N), s)


def _srcm(srcp, dstp):
    return pl.pallas_call(
        _srcm_body,
        out_shape=jax.ShapeDtypeStruct((EPAD // 128, 128), jnp.int32),
    )(srcp, dstp)


_STRIPE_CHUNKS = [(0, BE), (BE, BE), (2 * BE, BE), (3 * BE, BE),
                  (4 * BE, RPT - 4 * BE)]


@functools.partial(
    pl.kernel,
    mesh=_SC_MESH,
    out_type=jax.ShapeDtypeStruct((NCORE, NPAD, 128), jnp.float32),
    scratch_types=[
        pltpu.VMEM((BATCHES, BE), jnp.int32),
        pltpu.VMEM((BE, 128), jnp.float32),
        pltpu.VMEM_SHARED((NPAD, 128), jnp.float32),
        pltpu.SemaphoreType.DMA,
    ],
)
def _deg_kernel(srcm_hbm, out_hbm, sidx, rows, acc, sem):
    c_ax = lax.axis_index("c")
    s_ax = lax.axis_index("s")
    wid = c_ax * NSUB + s_ax
    pltpu.sync_copy(srcm_hbm.at[wid], sidx)
    zero = jnp.zeros((16,), jnp.float32)
    one = jnp.full((16,), 1.0, jnp.float32)

    def fz(i, carry):
        for j in range(8):
            rows[i, pl.ds(j * 16, 16)] = zero
        return carry

    lax.fori_loop(0, BE, fz, 0)
    rbase = s_ax * RPT
    for off, nrow in _STRIPE_CHUNKS:
        pltpu.sync_copy(rows.at[pl.ds(0, nrow)], acc.at[pl.ds(rbase + off, nrow)])

    def fo(i, carry):
        for j in range(8):
            rows[i, pl.ds(j * 16, 16)] = one
        return carry

    lax.fori_loop(0, BE, fo, 0)
    plsc.subcore_barrier()

    def eb(b, carry):
        pltpu.async_copy(rows, acc.at[sidx.at[b]], sem, add=True)
        return carry

    lax.fori_loop(0, BATCHES, eb, 0)

    def dr(b, carry):
        pltpu.make_async_copy(rows, acc.at[sidx.at[b]], sem).wait()
        return carry

    lax.fori_loop(0, BATCHES, dr, 0)
    plsc.subcore_barrier()
    for off, nrow in _STRIPE_CHUNKS:
        pltpu.sync_copy(acc.at[pl.ds(rbase + off, nrow)],
                        out_hbm.at[c_ax, pl.ds(rbase + off, nrow)])


def _make_prop(C):
    @functools.partial(
        pl.kernel,
        mesh=_SC_MESH,
        out_type=jax.ShapeDtypeStruct((NCORE, C, NPAD, 128), jnp.float32),
        scratch_types=[
            pltpu.VMEM((BATCHES, BE), jnp.int32),
            pltpu.VMEM((BATCHES, BE), jnp.int32),
            pltpu.VMEM((BE, 128), jnp.float32),
            pltpu.VMEM((BE, 128), jnp.float32),
            pltpu.VMEM_SHARED((NPAD, 128), jnp.float32),
            pltpu.SemaphoreType.DMA,
            pltpu.SemaphoreType.DMA,
        ],
    )
    def prop(g_hbm, srcm_hbm, dst_hbm, out_hbm, sidx, didx, rows_a, rows_b,
             acc, sem_a, sem_b):
        c_ax = lax.axis_index("c")
        s_ax = lax.axis_index("s")
        wid = c_ax * NSUB + s_ax
        pltpu.sync_copy(srcm_hbm.at[wid], sidx)
        pltpu.sync_copy(dst_hbm.at[wid], didx)
        zero = jnp.zeros((16,), jnp.float32)

        def zf(i, carry):
            for j in range(8):
                rows_a[i, pl.ds(j * 16, 16)] = zero
            return carry

        rbase = s_ax * RPT
        for c in range(C):
            gc = g_hbm.at[c]
            lax.fori_loop(0, BE, zf, 0)
            for off, nrow in _STRIPE_CHUNKS:
                pltpu.sync_copy(rows_a.at[pl.ds(0, nrow)],
                                acc.at[pl.ds(rbase + off, nrow)])
            pltpu.async_copy(gc.at[sidx.at[0]], rows_a, sem_a)
            pltpu.async_copy(gc.at[sidx.at[1]], rows_b, sem_b)
            plsc.subcore_barrier()

            def eb(k, carry):
                b = 2 * k
                pltpu.make_async_copy(gc.at[sidx.at[b]], rows_a, sem_a).wait()
                pltpu.sync_copy(rows_a, acc.at[didx.at[b]], add=True)

                @pl.when(b + 2 < BATCHES)
                def _():
                    pltpu.async_copy(gc.at[sidx.at[b + 2]], rows_a, sem_a)

                pltpu.make_async_copy(gc.at[sidx.at[b + 1]], rows_b, sem_b).wait()
                pltpu.sync_copy(rows_b, acc.at[didx.at[b + 1]], add=True)

                @pl.when(b + 3 < BATCHES)
                def _():
                    pltpu.async_copy(gc.at[sidx.at[b + 3]], rows_b, sem_b)

                return carry

            lax.fori_loop(0, BATCHES // 2, eb, 0)
            plsc.subcore_barrier()
            for off, nrow in _STRIPE_CHUNKS:
                pltpu.sync_copy(acc.at[pl.ds(rbase + off, nrow)],
                                out_hbm.at[c_ax, c, pl.ds(rbase + off, nrow)])

    return prop


_PROP = {c: _make_prop(c) for c in (1,)}


def _make_prop2x(C):
    C2 = max(C // 2, 1)

    @functools.partial(
        pl.kernel,
        mesh=_SC_MESH,
        out_type=[
            jax.ShapeDtypeStruct((C, NPAD, 128), jnp.float32),
            jax.ShapeDtypeStruct((C, NPAD, 128), jnp.float32),
            jax.ShapeDtypeStruct((C, NPAD, 128), jnp.float32),
        ],
        scratch_types=[
            pltpu.VMEM((BATCHES, BE), jnp.int32),
            pltpu.VMEM((BATCHES, BE), jnp.int32),
            pltpu.VMEM((BE, 128), jnp.float32),
            pltpu.VMEM((BE, 128), jnp.float32),
            pltpu.VMEM_SHARED((NPAD, 128), jnp.float32),
            pltpu.SemaphoreType.DMA,
            pltpu.SemaphoreType.DMA,
        ],
    )
    def prop2x(g_hbm, srcm_hbm, dst_hbm, dinv2_hbm, s1_hbm, g1_hbm, s2_hbm,
               sidx, didx, rows_a, rows_b, acc, sem_a, sem_b):
        c_ax = lax.axis_index("c")
        s_ax = lax.axis_index("s")
        zero = jnp.zeros((16,), jnp.float32)

        def zf(i, carry):
            for j in range(8):
                rows_a[i, pl.ds(j * 16, 16)] = zero
            return carry

        rbase = s_ax * RPT

        def zero_stripe():
            lax.fori_loop(0, BE, zf, 0)
            for off, nrow in _STRIPE_CHUNKS:
                pltpu.sync_copy(rows_a.at[pl.ds(0, nrow)],
                                acc.at[pl.ds(rbase + off, nrow)])

        def edge_pass(table):
            for h in range(2):
                base = (2 * s_ax + h) * BATCHES
                pltpu.sync_copy(srcm_hbm.at[pl.ds(base, BATCHES)], sidx)
                pltpu.sync_copy(dst_hbm.at[pl.ds(base, BATCHES)], didx)
                pltpu.async_copy(table.at[sidx.at[0]], rows_a, sem_a)
                pltpu.async_copy(table.at[sidx.at[1]], rows_b, sem_b)

                def eb(k, carry):
                    b = 2 * k
                    pltpu.make_async_copy(table.at[sidx.at[b]], rows_a, sem_a).wait()
                    pltpu.sync_copy(rows_a, acc.at[didx.at[b]], add=True)

                    @pl.when(b + 2 < BATCHES)
                    def _():
                        pltpu.async_copy(table.at[sidx.at[b + 2]], rows_a, sem_a)

                    pltpu.make_async_copy(table.at[sidx.at[b + 1]], rows_b, sem_b).wait()
                    pltpu.sync_copy(rows_b, acc.at[didx.at[b + 1]], add=True)

                    @pl.when(b + 3 < BATCHES)
                    def _():
                        pltpu.async_copy(table.at[sidx.at[b + 3]], rows_b, sem_b)

                    return carry

                lax.fori_loop(0, BATCHES // 2, eb, 0)

        def vmulneg(nrow):
            def vm(i, carry):
                for j in range(8):
                    sl = pl.ds(j * 16, 16)
                    rows_a[i, sl] = -(rows_a[i, sl] * rows_b[i, sl])
                return carry

            lax.fori_loop(0, nrow, vm, 0)

        for cc in range(C2):
            k = cc if C == 1 else c_ax * C2 + cc
            zero_stripe()
            plsc.subcore_barrier()
            edge_pass(g_hbm.at[k])
            plsc.subcore_barrier()
            for off, nrow in _STRIPE_CHUNKS:
                pltpu.sync_copy(acc.at[pl.ds(rbase + off, nrow)],
                                s1_hbm.at[k, pl.ds(rbase + off, nrow)])
                pltpu.sync_copy(acc.at[pl.ds(rbase + off, nrow)],
                                rows_a.at[pl.ds(0, nrow)])
                pltpu.sync_copy(dinv2_hbm.at[pl.ds(rbase + off, nrow)],
                                rows_b.at[pl.ds(0, nrow)])
                vmulneg(nrow)
                pltpu.sync_copy(rows_a.at[pl.ds(0, nrow)],
                                g1_hbm.at[k, pl.ds(rbase + off, nrow)])
            zero_stripe()
            plsc.subcore_barrier()
            edge_pass(g1_hbm.at[k])
            plsc.subcore_barrier()
            for off, nrow in _STRIPE_CHUNKS:
                pltpu.sync_copy(acc.at[pl.ds(rbase + off, nrow)],
                                s2_hbm.at[k, pl.ds(rbase + off, nrow)])

    return prop2x


_PROP2X = {c: _make_prop2x(c) for c in (1, 2, 4)}


def _dinv_g0_body(degp_ref, x_ref, dinv_ref, dinv2_ref, g0_ref):
    i = pl.program_id(0)
    deg = degp_ref[0, :, 0:1] + degp_ref[1, :, 0:1]
    rid = i * RBS + lax.broadcasted_iota(jnp.int32, (RBS, 1), 0)
    dinv = jnp.where((deg > 0.0) & (rid < N), lax.rsqrt(deg), 0.0)
    db = jnp.broadcast_to(dinv, (RBS, 128))
    dinv_ref[...] = db
    dinv2_ref[...] = db * db
    g0_ref[0] = db * x_ref[...]


def _dinv_g0(degp, xpad):
    return pl.pallas_call(
        _dinv_g0_body,
        grid=(RB,),
        in_specs=[
            pl.BlockSpec((NCORE, RBS, 128), lambda i: (0, i, 0)),
            pl.BlockSpec((RBS, 128), lambda i: (i, 0)),
        ],
        out_specs=[
            pl.BlockSpec((RBS, 128), lambda i: (i, 0)),
            pl.BlockSpec((RBS, 128), lambda i: (i, 0)),
            pl.BlockSpec((1, RBS, 128), lambda i: (0, i, 0)),
        ],
        out_shape=[
            jax.ShapeDtypeStruct((NPAD, 128), jnp.float32),
            jax.ShapeDtypeStruct((NPAD, 128), jnp.float32),
            jax.ShapeDtypeStruct((1, NPAD, 128), jnp.float32),
        ],
    )(degp, xpad)


def _combine_body(p_ref, dinv_ref, tx1_ref, g1_ref):
    s = p_ref[0, 0] + p_ref[1, 0]
    d = dinv_ref[...]
    t = -d * s
    tx1_ref[0] = t
    g1_ref[0] = d * t


def _combine(p1, dinv, C):
    return pl.pallas_call(
        _combine_body,
        grid=(RB, C),
        in_specs=[
            pl.BlockSpec((NCORE, 1, RBS, 128), lambda i, c: (0, c, i, 0)),
            pl.BlockSpec((RBS, 128), lambda i, c: (i, 0)),
        ],
        out_specs=[
            pl.BlockSpec((1, RBS, 128), lambda i, c: (c, i, 0)),
            pl.BlockSpec((1, RBS, 128), lambda i, c: (c, i, 0)),
        ],
        out_shape=[
            jax.ShapeDtypeStruct((C, NPAD, 128), jnp.float32),
            jax.ShapeDtypeStruct((C, NPAD, 128), jnp.float32),
        ],
    )(p1, dinv)


def _make_mm_body(C, COB):
    MPB = COB * 128

    def body(x0_ref, tx1_ref, p2_ref, dinv_ref, w0_ref, w1_ref, w2_ref, b_ref,
             out_ref, g_ref):
        d = dinv_ref[...]
        acc = jnp.broadcast_to(b_ref[...], (RBS, MPB))
        for c in range(C):
            x0 = x0_ref[c]
            x1 = tx1_ref[c]
            x2 = -2.0 * d * (p2_ref[0, c] + p2_ref[1, c]) - x0
            acc = acc + jnp.dot(x0, w0_ref[c], preferred_element_type=jnp.float32)
            acc = acc + jnp.dot(x1, w1_ref[c], preferred_element_type=jnp.float32)
            acc = acc + jnp.dot(x2, w2_ref[c], preferred_element_type=jnp.float32)
        h = jnp.maximum(acc, 0.0)
        for co in range(COB):
            hc = h[:, co * 128:(co + 1) * 128]
            out_ref[co] = hc
            g_ref[co] = d * hc

    return body


def _make_mm2_body(C, COB):
    MPB = COB * 128

    def body(x0_ref, s1_ref, s2_ref, dinv_ref, w0_ref, w1_ref, w2_ref, b_ref,
             out_ref, g_ref):
        d = dinv_ref[...]
        acc = jnp.broadcast_to(b_ref[...], (RBS, MPB))
        for c in range(C):
            x0 = x0_ref[c]
            x1 = -d * s1_ref[c]
            x2 = -2.0 * d * s2_ref[c] - x0
            acc = acc + jnp.dot(x0, w0_ref[c], preferred_element_type=jnp.float32)
            acc = acc + jnp.dot(x1, w1_ref[c], preferred_element_type=jnp.float32)
            acc = acc + jnp.dot(x2, w2_ref[c], preferred_element_type=jnp.float32)
        h = jnp.maximum(acc, 0.0)
        for co in range(COB):
            hc = h[:, co * 128:(co + 1) * 128]
            out_ref[co] = hc
            g_ref[co] = d * hc

    return body


def _layer_mm2(x0, s1, s2, dinv, w0, w1, w2, b, C, CO, msplit):
    COB = CO // msplit
    MPB = COB * 128
    body = _make_mm2_body(C, COB)
    return pl.pallas_call(
        body,
        grid=(RB, msplit),
        in_specs=[
            pl.BlockSpec((C, RBS, 128), lambda i, m: (0, i, 0)),
            pl.BlockSpec((C, RBS, 128), lambda i, m: (0, i, 0)),
            pl.BlockSpec((C, RBS, 128), lambda i, m: (0, i, 0)),
            pl.BlockSpec((RBS, 128), lambda i, m: (i, 0)),
            pl.BlockSpec((C, 128, MPB), lambda i, m: (0, 0, m)),
            pl.BlockSpec((C, 128, MPB), lambda i, m: (0, 0, m)),
            pl.BlockSpec((C, 128, MPB), lambda i, m: (0, 0, m)),
            pl.BlockSpec((1, MPB), lambda i, m: (0, m)),
        ],
        out_specs=[
            pl.BlockSpec((COB, RBS, 128), lambda i, m: (m, i, 0)),
            pl.BlockSpec((COB, RBS, 128), lambda i, m: (m, i, 0)),
        ],
        out_shape=[
            jax.ShapeDtypeStruct((CO, NPAD, 128), jnp.float32),
            jax.ShapeDtypeStruct((CO, NPAD, 128), jnp.float32),
        ],
    )(x0, s1, s2, dinv, w0, w1, w2, b)


def _layer_mm(x0, tx1, p2, dinv, w0, w1, w2, b, C, CO, msplit):
    COB = CO // msplit
    MPB = COB * 128
    body = _make_mm_body(C, COB)
    return pl.pallas_call(
        body,
        grid=(RB, msplit),
        in_specs=[
            pl.BlockSpec((C, RBS, 128), lambda i, m: (0, i, 0)),
            pl.BlockSpec((C, RBS, 128), lambda i, m: (0, i, 0)),
            pl.BlockSpec((NCORE, C, RBS, 128), lambda i, m: (0, 0, i, 0)),
            pl.BlockSpec((RBS, 128), lambda i, m: (i, 0)),
            pl.BlockSpec((C, 128, MPB), lambda i, m: (0, 0, m)),
            pl.BlockSpec((C, 128, MPB), lambda i, m: (0, 0, m)),
            pl.BlockSpec((C, 128, MPB), lambda i, m: (0, 0, m)),
            pl.BlockSpec((1, MPB), lambda i, m: (0, m)),
        ],
        out_specs=[
            pl.BlockSpec((COB, RBS, 128), lambda i, m: (m, i, 0)),
            pl.BlockSpec((COB, RBS, 128), lambda i, m: (m, i, 0)),
        ],
        out_shape=[
            jax.ShapeDtypeStruct((CO, NPAD, 128), jnp.float32),
            jax.ShapeDtypeStruct((CO, NPAD, 128), jnp.float32),
        ],
    )(x0, tx1, p2, dinv, w0, w1, w2, b)


def _prep_w(w, C, MP):
    return jnp.pad(w, ((0, C * 128 - w.shape[0]), (0, MP - w.shape[1]))).reshape(C, 128, MP)


def _prep_ws(w0, w1, w2, b, C, CO):
    MP = CO * 128
    return (_prep_w(w0, C, MP), _prep_w(w1, C, MP), _prep_w(w2, C, MP),
            jnp.pad(b, (0, MP - b.shape[0])).reshape(1, MP))


def _layer2x(h_ch, g_ch, dinv, dinv2, srcm2, dst2, w0, w1, w2, b, C, CO, msplit):
    w0c, w1c, w2c, bp = _prep_ws(w0, w1, w2, b, C, CO)
    s1, _g1, s2 = _PROP2X[C](g_ch, srcm2, dst2, dinv2)
    return _layer_mm2(h_ch, s1, s2, dinv, w0c, w1c, w2c, bp, C, CO, msplit)


def kernel(x, edge_index, W1_0, W1_1, W1_2, b1, W2_0, W2_1, W2_2, b2,
           W3_0, W3_1, W3_2, b3):
    src = edge_index[0]
    dst = edge_index[1]
    padi = jnp.zeros((EPAD - E,), jnp.int32)
    srcp = jnp.concatenate([src, padi]).reshape(EPAD // 128, 128)
    dstp = jnp.concatenate([dst, padi]).reshape(EPAD // 128, 128)
    srcm2 = _srcm(srcp, dstp)
    srcm3 = srcm2.reshape(NTILES, BATCHES, BE)
    dst3 = dstp.reshape(NTILES, BATCHES, BE)
    degp = _deg_kernel(srcm3)
    xpad = jnp.pad(x, ((0, NPAD - N), (0, 0)))
    dinv, dinv2, g0 = _dinv_g0(degp, xpad)
    h0 = xpad.reshape(1, NPAD, 128)
    h1, g1n = _layer2x(h0, g0, dinv, dinv2, srcm2, dstp, W1_0, W1_1, W1_2,
                       b1, 1, 2, 1)
    h2, g2n = _layer2x(h1, g1n, dinv, dinv2, srcm2, dstp, W2_0, W2_1, W2_2,
                       b2, 2, 4, 1)
    h3, _ = _layer2x(h2, g2n, dinv, dinv2, srcm2, dstp, W3_0, W3_1, W3_2,
                     b3, 4, 8, 2)
    out = jnp.transpose(h3, (1, 0, 2)).reshape(NPAD, 1024)
    return out[:N, :1000]


# speedup vs baseline: 1.1124x; 1.1124x over previous
import functools

import jax
import jax.numpy as jnp
from jax import lax
from jax.experimental import pallas as pl
from jax.experimental.pallas import tpu as pltpu
from jax.experimental.pallas import tpu_sc as plsc

N = 10000
NPAD = 10112
E = 160000
EPAD = 163840
NTILES = 32
NSUB = 16
NCORE = 2
BATCHES = EPAD // (NTILES * 128)
BE = 128
RPT = NPAD // NSUB
RB = 8
RBS = NPAD // RB

_SC_MESH = plsc.VectorSubcoreMesh(core_axis_name="c", subcore_axis_name="s")


def _srcm_body(src_ref, dst_ref, o_ref):
    s = src_ref[...]
    d = dst_ref[...]
    o_ref[...] = jnp.where(s == d, jnp.int32(N), s)


def _srcm(srcp, dstp):
    return pl.pallas_call(
        _srcm_body,
        out_shape=jax.ShapeDtypeStruct((EPAD // 128, 128), jnp.int32),
    )(srcp, dstp)


_STRIPE_CHUNKS = [(0, BE), (BE, BE), (2 * BE, BE), (3 * BE, BE),
                  (4 * BE, RPT - 4 * BE)]


@functools.partial(
    pl.kernel,
    mesh=_SC_MESH,
    out_type=jax.ShapeDtypeStruct((NCORE, NPAD, 128), jnp.float32),
    scratch_types=[
        pltpu.VMEM((BATCHES, BE), jnp.int32),
        pltpu.VMEM((BE, 128), jnp.float32),
        pltpu.VMEM_SHARED((NPAD, 128), jnp.float32),
        pltpu.SemaphoreType.DMA,
    ],
)
def _deg_kernel(srcm_hbm, out_hbm, sidx, rows, acc, sem):
    c_ax = lax.axis_index("c")
    s_ax = lax.axis_index("s")
    wid = c_ax * NSUB + s_ax
    pltpu.sync_copy(srcm_hbm.at[wid], sidx)
    zero = jnp.zeros((16,), jnp.float32)
    one = jnp.full((16,), 1.0, jnp.float32)

    def fz(i, carry):
        for j in range(8):
            rows[i, pl.ds(j * 16, 16)] = zero
        return carry

    lax.fori_loop(0, BE, fz, 0)
    rbase = s_ax * RPT
    for off, nrow in _STRIPE_CHUNKS:
        pltpu.sync_copy(rows.at[pl.ds(0, nrow)], acc.at[pl.ds(rbase + off, nrow)])

    def fo(i, carry):
        for j in range(8):
            rows[i, pl.ds(j * 16, 16)] = one
        return carry

    lax.fori_loop(0, BE, fo, 0)
    plsc.subcore_barrier()

    def eb(b, carry):
        pltpu.async_copy(rows, acc.at[sidx.at[b]], sem, add=True)
        return carry

    lax.fori_loop(0, BATCHES, eb, 0)

    def dr(b, carry):
        pltpu.make_async_copy(rows, acc.at[sidx.at[b]], sem).wait()
        return carry

    lax.fori_loop(0, BATCHES, dr, 0)
    plsc.subcore_barrier()
    for off, nrow in _STRIPE_CHUNKS:
        pltpu.sync_copy(acc.at[pl.ds(rbase + off, nrow)],
                        out_hbm.at[c_ax, pl.ds(rbase + off, nrow)])


def _make_prop(C):
    @functools.partial(
        pl.kernel,
        mesh=_SC_MESH,
        out_type=jax.ShapeDtypeStruct((NCORE, C, NPAD, 128), jnp.float32),
        scratch_types=[
            pltpu.VMEM((BATCHES, BE), jnp.int32),
            pltpu.VMEM((BATCHES, BE), jnp.int32),
            pltpu.VMEM((BE, 128), jnp.float32),
            pltpu.VMEM((BE, 128), jnp.float32),
            pltpu.VMEM_SHARED((NPAD, 128), jnp.float32),
            pltpu.SemaphoreType.DMA,
            pltpu.SemaphoreType.DMA,
        ],
    )
    def prop(g_hbm, srcm_hbm, dst_hbm, out_hbm, sidx, didx, rows_a, rows_b,
             acc, sem_a, sem_b):
        c_ax = lax.axis_index("c")
        s_ax = lax.axis_index("s")
        wid = c_ax * NSUB + s_ax
        pltpu.sync_copy(srcm_hbm.at[wid], sidx)
        pltpu.sync_copy(dst_hbm.at[wid], didx)
        zero = jnp.zeros((16,), jnp.float32)

        def zf(i, carry):
            for j in range(8):
                rows_a[i, pl.ds(j * 16, 16)] = zero
            return carry

        rbase = s_ax * RPT
        for c in range(C):
            gc = g_hbm.at[c]
            lax.fori_loop(0, BE, zf, 0)
            for off, nrow in _STRIPE_CHUNKS:
                pltpu.sync_copy(rows_a.at[pl.ds(0, nrow)],
                                acc.at[pl.ds(rbase + off, nrow)])
            pltpu.async_copy(gc.at[sidx.at[0]], rows_a, sem_a)
            pltpu.async_copy(gc.at[sidx.at[1]], rows_b, sem_b)
            plsc.subcore_barrier()

            def eb(k, carry):
                b = 2 * k
                pltpu.make_async_copy(gc.at[sidx.at[b]], rows_a, sem_a).wait()
                pltpu.sync_copy(rows_a, acc.at[didx.at[b]], add=True)

                @pl.when(b + 2 < BATCHES)
                def _():
                    pltpu.async_copy(gc.at[sidx.at[b + 2]], rows_a, sem_a)

                pltpu.make_async_copy(gc.at[sidx.at[b + 1]], rows_b, sem_b).wait()
                pltpu.sync_copy(rows_b, acc.at[didx.at[b + 1]], add=True)

                @pl.when(b + 3 < BATCHES)
                def _():
                    pltpu.async_copy(gc.at[sidx.at[b + 3]], rows_b, sem_b)

                return carry

            lax.fori_loop(0, BATCHES // 2, eb, 0)
            plsc.subcore_barrier()
            for off, nrow in _STRIPE_CHUNKS:
                pltpu.sync_copy(acc.at[pl.ds(rbase + off, nrow)],
                                out_hbm.at[c_ax, c, pl.ds(rbase + off, nrow)])

    return prop


_PROP = {c: _make_prop(c) for c in (1,)}


def _make_prop2x(C):
    C2 = max(C // 2, 1)

    @functools.partial(
        pl.kernel,
        mesh=_SC_MESH,
        out_type=[
            jax.ShapeDtypeStruct((C, NPAD, 128), jnp.float32),
            jax.ShapeDtypeStruct((C, NPAD, 128), jnp.float32),
            jax.ShapeDtypeStruct((C, NPAD, 128), jnp.float32),
        ],
        scratch_types=[
            pltpu.VMEM((BATCHES, BE), jnp.int32),
            pltpu.VMEM((BATCHES, BE), jnp.int32),
            pltpu.VMEM((BE, 128), jnp.float32),
            pltpu.VMEM((BE, 128), jnp.float32),
            pltpu.VMEM_SHARED((NPAD, 128), jnp.float32),
            pltpu.SemaphoreType.DMA,
            pltpu.SemaphoreType.DMA,
        ],
    )
    def prop2x(g_hbm, srcm_hbm, dst_hbm, dinv2_hbm, s1_hbm, g1_hbm, s2_hbm,
               sidx, didx, rows_a, rows_b, acc, sem_a, sem_b):
        c_ax = lax.axis_index("c")
        s_ax = lax.axis_index("s")
        zero = jnp.zeros((16,), jnp.float32)

        def zf(i, carry):
            for j in range(8):
                rows_a[i, pl.ds(j * 16, 16)] = zero
            return carry

        rbase = s_ax * RPT

        def zero_stripe():
            lax.fori_loop(0, BE, zf, 0)
            for off, nrow in _STRIPE_CHUNKS:
                pltpu.sync_copy(rows_a.at[pl.ds(0, nrow)],
                                acc.at[pl.ds(rbase + off, nrow)])

        def edge_pass(table):
            for h in range(2):
                base = (2 * s_ax + h) * BATCHES
                pltpu.sync_copy(srcm_hbm.at[pl.ds(base, BATCHES)], sidx)
                pltpu.sync_copy(dst_hbm.at[pl.ds(base, BATCHES)], didx)
                pltpu.async_copy(table.at[sidx.at[0]], rows_a, sem_a)
                pltpu.async_copy(table.at[sidx.at[1]], rows_b, sem_b)

                def eb(k, carry):
                    b = 2 * k
                    pltpu.make_async_copy(table.at[sidx.at[b]], rows_a, sem_a).wait()
                    pltpu.sync_copy(rows_a, acc.at[didx.at[b]], add=True)

                    @pl.when(b + 2 < BATCHES)
                    def _():
                        pltpu.async_copy(table.at[sidx.at[b + 2]], rows_a, sem_a)

                    pltpu.make_async_copy(table.at[sidx.at[b + 1]], rows_b, sem_b).wait()
                    pltpu.sync_copy(rows_b, acc.at[didx.at[b + 1]], add=True)

                    @pl.when(b + 3 < BATCHES)
                    def _():
                        pltpu.async_copy(table.at[sidx.at[b + 3]], rows_b, sem_b)

                    return carry

                lax.fori_loop(0, BATCHES // 2, eb, 0)

        def vmulneg(nrow):
            def vm(i, carry):
                for j in range(8):
                    sl = pl.ds(j * 16, 16)
                    rows_a[i, sl] = -(rows_a[i, sl] * rows_b[i, sl])
                return carry

            lax.fori_loop(0, nrow, vm, 0)

        for cc in range(C2):
            k = cc if C == 1 else c_ax * C2 + cc
            zero_stripe()
            plsc.subcore_barrier()
            edge_pass(g_hbm.at[k])
            plsc.subcore_barrier()
            for off, nrow in _STRIPE_CHUNKS:
                pltpu.sync_copy(acc.at[pl.ds(rbase + off, nrow)],
                                s1_hbm.at[k, pl.ds(rbase + off, nrow)])
                pltpu.sync_copy(acc.at[pl.ds(rbase + off, nrow)],
                                rows_a.at[pl.ds(0, nrow)])
                pltpu.sync_copy(dinv2_hbm.at[pl.ds(rbase + off, nrow)],
                                rows_b.at[pl.ds(0, nrow)])
                vmulneg(nrow)
                pltpu.sync_copy(rows_a.at[pl.ds(0, nrow)],
                                g1_hbm.at[k, pl.ds(rbase + off, nrow)])
            zero_stripe()
            plsc.subcore_barrier()
            edge_pass(g1_hbm.at[k])
            plsc.subcore_barrier()
            for off, nrow in _STRIPE_CHUNKS:
                pltpu.sync_copy(acc.at[pl.ds(rbase + off, nrow)],
                                s2_hbm.at[k, pl.ds(rbase + off, nrow)])

    return prop2x


_PROP2X = {c: _make_prop2x(c) for c in (1, 2, 4)}


def _dinv_g0_body(degp_ref, x_ref, dinv_ref, dinv2_ref, g0_ref):
    i = pl.program_id(0)
    deg = degp_ref[0, :, 0:1] + degp_ref[1, :, 0:1]
    rid = i * RBS + lax.broadcasted_iota(jnp.int32, (RBS, 1), 0)
    dinv = jnp.where((deg > 0.0) & (rid < N), lax.rsqrt(deg), 0.0)
    db = jnp.broadcast_to(dinv, (RBS, 128))
    dinv_ref[...] = db
    dinv2_ref[...] = db * db
    g0_ref[0] = db * x_ref[...]


def _dinv_g0(degp, xpad):
    return pl.pallas_call(
        _dinv_g0_body,
        grid=(RB,),
        in_specs=[
            pl.BlockSpec((NCORE, RBS, 128), lambda i: (0, i, 0)),
            pl.BlockSpec((RBS, 128), lambda i: (i, 0)),
        ],
        out_specs=[
            pl.BlockSpec((RBS, 128), lambda i: (i, 0)),
            pl.BlockSpec((RBS, 128), lambda i: (i, 0)),
            pl.BlockSpec((1, RBS, 128), lambda i: (0, i, 0)),
        ],
        out_shape=[
            jax.ShapeDtypeStruct((NPAD, 128), jnp.float32),
            jax.ShapeDtypeStruct((NPAD, 128), jnp.float32),
            jax.ShapeDtypeStruct((1, NPAD, 128), jnp.float32),
        ],
    )(degp, xpad)


def _combine_body(p_ref, dinv_ref, tx1_ref, g1_ref):
    s = p_ref[0, 0] + p_ref[1, 0]
    d = dinv_ref[...]
    t = -d * s
    tx1_ref[0] = t
    g1_ref[0] = d * t


def _combine(p1, dinv, C):
    return pl.pallas_call(
        _combine_body,
        grid=(RB, C),
        in_specs=[
            pl.BlockSpec((NCORE, 1, RBS, 128), lambda i, c: (0, c, i, 0)),
            pl.BlockSpec((RBS, 128), lambda i, c: (i, 0)),
        ],
        out_specs=[
            pl.BlockSpec((1, RBS, 128), lambda i, c: (c, i, 0)),
            pl.BlockSpec((1, RBS, 128), lambda i, c: (c, i, 0)),
        ],
        out_shape=[
            jax.ShapeDtypeStruct((C, NPAD, 128), jnp.float32),
            jax.ShapeDtypeStruct((C, NPAD, 128), jnp.float32),
        ],
    )(p1, dinv)


def _make_mm_body(C, COB):
    MPB = COB * 128

    def body(x0_ref, tx1_ref, p2_ref, dinv_ref, w0_ref, w1_ref, w2_ref, b_ref,
             out_ref, g_ref):
        d = dinv_ref[...]
        acc = jnp.broadcast_to(b_ref[...], (RBS, MPB))
        for c in range(C):
            x0 = x0_ref[c]
            x1 = tx1_ref[c]
            x2 = -2.0 * d * (p2_ref[0, c] + p2_ref[1, c]) - x0
            acc = acc + jnp.dot(x0, w0_ref[c], preferred_element_type=jnp.float32)
            acc = acc + jnp.dot(x1, w1_ref[c], preferred_element_type=jnp.float32)
            acc = acc + jnp.dot(x2, w2_ref[c], preferred_element_type=jnp.float32)
        h = jnp.maximum(acc, 0.0)
        for co in range(COB):
            hc = h[:, co * 128:(co + 1) * 128]
            out_ref[co] = hc
            g_ref[co] = d * hc

    return body


def _make_mm2_body(C, COB):
    MPB = COB * 128

    def body(x0_ref, s1_ref, s2_ref, dinv_ref, w0_ref, w1_ref, w2_ref, b_ref,
             out_ref, g_ref):
        d = dinv_ref[...]
        acc = jnp.broadcast_to(b_ref[...], (RBS, MPB))
        for c in range(C):
            x0 = x0_ref[c]
            x1 = -d * s1_ref[c]
            x2 = -2.0 * d * s2_ref[c] - x0
            acc = acc + jnp.dot(x0, w0_ref[c], preferred_element_type=jnp.float32)
            acc = acc + jnp.dot(x1, w1_ref[c], preferred_element_type=jnp.float32)
            acc = acc + jnp.dot(x2, w2_ref[c], preferred_element_type=jnp.float32)
        h = jnp.maximum(acc, 0.0)
        for co in range(COB):
            hc = h[:, co * 128:(co + 1) * 128]
            out_ref[co] = hc
            g_ref[co] = d * hc

    return body


def _layer_mm2(x0, s1, s2, dinv, w0, w1, w2, b, C, CO, msplit):
    COB = CO // msplit
    MPB = COB * 128
    body = _make_mm2_body(C, COB)
    return pl.pallas_call(
        body,
        grid=(RB, msplit),
        in_specs=[
            pl.BlockSpec((C, RBS, 128), lambda i, m: (0, i, 0)),
            pl.BlockSpec((C, RBS, 128), lambda i, m: (0, i, 0)),
            pl.BlockSpec((C, RBS, 128), lambda i, m: (0, i, 0)),
            pl.BlockSpec((RBS, 128), lambda i, m: (i, 0)),
            pl.BlockSpec((C, 128, MPB), lambda i, m: (0, 0, m)),
            pl.BlockSpec((C, 128, MPB), lambda i, m: (0, 0, m)),
            pl.BlockSpec((C, 128, MPB), lambda i, m: (0, 0, m)),
            pl.BlockSpec((1, MPB), lambda i, m: (0, m)),
        ],
        out_specs=[
            pl.BlockSpec((COB, RBS, 128), lambda i, m: (m, i, 0)),
            pl.BlockSpec((COB, RBS, 128), lambda i, m: (m, i, 0)),
        ],
        out_shape=[
            jax.ShapeDtypeStruct((CO, NPAD, 128), jnp.float32),
            jax.ShapeDtypeStruct((CO, NPAD, 128), jnp.float32),
        ],
    )(x0, s1, s2, dinv, w0, w1, w2, b)


def _layer_mm(x0, tx1, p2, dinv, w0, w1, w2, b, C, CO, msplit):
    COB = CO // msplit
    MPB = COB * 128
    body = _make_mm_body(C, COB)
    return pl.pallas_call(
        body,
        grid=(RB, msplit),
        in_specs=[
            pl.BlockSpec((C, RBS, 128), lambda i, m: (0, i, 0)),
            pl.BlockSpec((C, RBS, 128), lambda i, m: (0, i, 0)),
            pl.BlockSpec((NCORE, C, RBS, 128), lambda i, m: (0, 0, i, 0)),
            pl.BlockSpec((RBS, 128), lambda i, m: (i, 0)),
            pl.BlockSpec((C, 128, MPB), lambda i, m: (0, 0, m)),
            pl.BlockSpec((C, 128, MPB), lambda i, m: (0, 0, m)),
            pl.BlockSpec((C, 128, MPB), lambda i, m: (0, 0, m)),
            pl.BlockSpec((1, MPB), lambda i, m: (0, m)),
        ],
        out_specs=[
            pl.BlockSpec((COB, RBS, 128), lambda i, m: (m, i, 0)),
            pl.BlockSpec((COB, RBS, 128), lambda i, m: (m, i, 0)),
        ],
        out_shape=[
            jax.ShapeDtypeStruct((CO, NPAD, 128), jnp.float32),
            jax.ShapeDtypeStruct((CO, NPAD, 128), jnp.float32),
        ],
    )(x0, tx1, p2, dinv, w0, w1, w2, b)


def _prep_w(w, C, MP):
    return jnp.pad(w, ((0, C * 128 - w.shape[0]), (0, MP - w.shape[1]))).reshape(C, 128, MP)


def _prep_ws(w0, w1, w2, b, C, CO):
    MP = CO * 128
    return (_prep_w(w0, C, MP), _prep_w(w1, C, MP), _prep_w(w2, C, MP),
            jnp.pad(b, (0, MP - b.shape[0])).reshape(1, MP))


def _layer1(h_ch, g_ch, dinv, srcm3, dst3, w0, w1, w2, b, CO):
    w0c, w1c, w2c, bp = _prep_ws(w0, w1, w2, b, 1, CO)
    p1 = _PROP[1](g_ch, srcm3, dst3)
    tx1, g1 = _combine(p1, dinv, 1)
    p2 = _PROP[1](g1, srcm3, dst3)
    return _layer_mm(h_ch, tx1, p2, dinv, w0c, w1c, w2c, bp, 1, CO, 1)


def _layer2x(h_ch, g_ch, dinv, dinv2, srcm2, dst2, w0, w1, w2, b, C, CO, msplit):
    w0c, w1c, w2c, bp = _prep_ws(w0, w1, w2, b, C, CO)
    s1, _g1, s2 = _PROP2X[C](g_ch, srcm2, dst2, dinv2)
    return _layer_mm2(h_ch, s1, s2, dinv, w0c, w1c, w2c, bp, C, CO, msplit)


def kernel(x, edge_index, W1_0, W1_1, W1_2, b1, W2_0, W2_1, W2_2, b2,
           W3_0, W3_1, W3_2, b3):
    src = edge_index[0]
    dst = edge_index[1]
    padi = jnp.zeros((EPAD - E,), jnp.int32)
    srcp = jnp.concatenate([src, padi]).reshape(EPAD // 128, 128)
    dstp = jnp.concatenate([dst, padi]).reshape(EPAD // 128, 128)
    srcm2 = _srcm(srcp, dstp)
    srcm3 = srcm2.reshape(NTILES, BATCHES, BE)
    dst3 = dstp.reshape(NTILES, BATCHES, BE)
    degp = _deg_kernel(srcm3)
    xpad = jnp.pad(x, ((0, NPAD - N), (0, 0)))
    dinv, dinv2, g0 = _dinv_g0(degp, xpad)
    h0 = xpad.reshape(1, NPAD, 128)
    h1, g1n = _layer1(h0, g0, dinv, srcm3, dst3, W1_0, W1_1, W1_2, b1, 2)
    h2, g2n = _layer2x(h1, g1n, dinv, dinv2, srcm2, dstp, W2_0, W2_1, W2_2,
                       b2, 2, 4, 1)
    h3, _ = _layer2x(h2, g2n, dinv, dinv2, srcm2, dstp, W3_0, W3_1, W3_2,
                     b3, 4, 8, 2)
    out = jnp.transpose(h3, (1, 0, 2)).reshape(NPAD, 1024)
    return out[:N, :1000]
